# Initial kernel scaffold; baseline (speedup 1.0000x reference)
import jax
import jax.numpy as jnp
from jax.experimental import pallas as pl


def kernel(x, W, b):
    raise NotImplementedError("write your pallas kernel here")


# speedup vs baseline: 52.8808x; 52.8808x over previous
import functools

import jax
import jax.numpy as jnp
from jax.experimental import pallas as pl

D_DENSE = 768
D_SPARSE = 1024
TOPK = 256
ROWS_PER_TILE = 512


def _tile_kernel(x_ref, w_ref, b_ref, o_ref):
    x = x_ref[...]
    w = w_ref[...]
    y = jax.lax.dot_general(
        x, w,
        dimension_numbers=(((1,), (1,)), ((), ())),
        preferred_element_type=jnp.float32,
    ) + b_ref[...]

    abits = jax.lax.bitcast_convert_type(jnp.abs(y), jnp.int32)

    def body(i, t):
        bit = 30 - i
        cand = t | (jnp.int32(1) << bit)
        cnt = jnp.sum((abits >= cand).astype(jnp.float32), axis=1,
                      keepdims=True)
        return jnp.where(cnt >= float(TOPK), cand, t)

    t0 = jnp.zeros((x.shape[0], 1), jnp.int32)
    thresh = jax.lax.fori_loop(0, 31, body, t0)
    o_ref[...] = jnp.where(abits >= thresh, y, 0.0)


@jax.jit
def kernel(x, W, b):
    B, S, _ = x.shape
    n = B * S
    x2 = x.reshape(n, D_DENSE)
    b2 = b.reshape(1, D_SPARSE)
    out = pl.pallas_call(
        _tile_kernel,
        grid=(n // ROWS_PER_TILE,),
        in_specs=[
            pl.BlockSpec((ROWS_PER_TILE, D_DENSE), lambda i: (i, 0)),
            pl.BlockSpec((D_SPARSE, D_DENSE), lambda i: (0, 0)),
            pl.BlockSpec((1, D_SPARSE), lambda i: (0, 0)),
        ],
        out_specs=pl.BlockSpec((ROWS_PER_TILE, D_SPARSE), lambda i: (i, 0)),
        out_shape=jax.ShapeDtypeStruct((n, D_SPARSE), jnp.float32),
    )(x2, W, b2)
    return out.reshape(B, S, D_SPARSE)


# speedup vs baseline: 52.9267x; 1.0009x over previous
import functools

import jax
import jax.numpy as jnp
from jax.experimental import pallas as pl
from jax.experimental.pallas import tpu as pltpu

D_DENSE = 768
D_SPARSE = 1024
TOPK = 256
ROWS_PER_TILE = 512


def _tile_kernel(x_ref, w_ref, b_ref, o_ref):
    x = x_ref[...]
    w = w_ref[...]
    y = jax.lax.dot_general(
        x, w,
        dimension_numbers=(((1,), (1,)), ((), ())),
        preferred_element_type=jnp.float32,
    ) + b_ref[...]

    abits = jax.lax.bitcast_convert_type(jnp.abs(y), jnp.int32)

    def body(i, t):
        bit = 30 - i
        cand = t | (jnp.int32(1) << bit)
        cnt = jnp.sum((abits >= cand).astype(jnp.float32), axis=1,
                      keepdims=True)
        return jnp.where(cnt >= float(TOPK), cand, t)

    t0 = jnp.zeros((x.shape[0], 1), jnp.int32)
    thresh = jax.lax.fori_loop(0, 31, body, t0)
    o_ref[...] = jnp.where(abits >= thresh, y, 0.0)


@jax.jit
def kernel(x, W, b):
    B, S, _ = x.shape
    n = B * S
    x2 = x.reshape(n, D_DENSE)
    b2 = b.reshape(1, D_SPARSE)
    out = pl.pallas_call(
        _tile_kernel,
        grid=(n // ROWS_PER_TILE,),
        in_specs=[
            pl.BlockSpec((ROWS_PER_TILE, D_DENSE), lambda i: (i, 0)),
            pl.BlockSpec((D_SPARSE, D_DENSE), lambda i: (0, 0)),
            pl.BlockSpec((1, D_SPARSE), lambda i: (0, 0)),
        ],
        out_specs=pl.BlockSpec((ROWS_PER_TILE, D_SPARSE), lambda i: (i, 0)),
        out_shape=jax.ShapeDtypeStruct((n, D_SPARSE), jnp.float32),
        compiler_params=pltpu.CompilerParams(
            dimension_semantics=("parallel",),
        ),
    )(x2, W, b2)
    return out.reshape(B, S, D_SPARSE)
